# Initial kernel scaffold; baseline (speedup 1.0000x reference)
import jax
import jax.numpy as jnp
from jax.experimental import pallas as pl


def kernel(feat_user, feat_item, edge_index_clicks, edge_index_clicked_by, params):
    raise NotImplementedError("write your pallas kernel here")


# speedup vs baseline: 27.8629x; 27.8629x over previous
import functools
import math

import jax
import jax.numpy as jnp
from jax import lax
from jax.experimental import pallas as pl
from jax.experimental.pallas import tpu as pltpu
from jax.experimental.pallas import tpu_sc as plsc

N_USER = 10000
N_ITEM = 10000
N_ALL = N_USER + N_ITEM
E = 160000
D = 128
H = 8
DK = 16
E_PAD = 163840
CHUNK = 128
PER_W = (2 * E_PAD) // 32
PER_T = E_PAD // 16
N_SUB = 16
ROWS_A = 624
ROWS_B = 640
RB_LAST = ROWS_A * (N_SUB - 1)


def _fold(w, b, wh, scale):
    wf = jnp.einsum("nhi,hij->nhj", w.reshape(D, H, DK), wh)
    bf = jnp.einsum("hi,hij->hj", b.reshape(H, DK), wh)
    if scale is not None:
        wf = wf * scale[None, :, None]
        bf = bf * scale[:, None]
    return wf.reshape(D, D), bf.reshape(D)


def _p1_body(f_ref, wq_ref, wkv_ref, bq_ref, bkv_ref, q_ref, kv_ref):
    f = f_ref[...]
    q_ref[...] = jnp.dot(f, wq_ref[0], preferred_element_type=jnp.float32) + bq_ref[0]
    kv_ref[...] = jnp.dot(f, wkv_ref[0], preferred_element_type=jnp.float32) + bkv_ref[0]


def _p1(feats, wq, wkv, bq, bkv, interpret=False):
    blk = 1000
    grid = (N_ALL // blk,)
    nt = lambda i: i // (N_USER // blk)
    return pl.pallas_call(
        _p1_body,
        grid=grid,
        in_specs=[
            pl.BlockSpec((blk, D), lambda i: (i, 0)),
            pl.BlockSpec((1, D, D), lambda i: (nt(i), 0, 0)),
            pl.BlockSpec((1, D, 2 * D), lambda i: (nt(i), 0, 0)),
            pl.BlockSpec((1, 1, D), lambda i: (nt(i), 0, 0)),
            pl.BlockSpec((1, 1, 2 * D), lambda i: (nt(i), 0, 0)),
        ],
        out_specs=[
            pl.BlockSpec((blk, D), lambda i: (i, 0)),
            pl.BlockSpec((blk, 2 * D), lambda i: (i, 0)),
        ],
        out_shape=[
            jax.ShapeDtypeStruct((N_ALL, D), jnp.float32),
            jax.ShapeDtypeStruct((N_ALL, 2 * D), jnp.float32),
        ],
        interpret=interpret,
    )(feats, wq, wkv, bq.reshape(2, 1, D), bkv.reshape(2, 1, 2 * D))


def _sc_gather(q_tab, kv_tab, dst_g, src_g):
    mesh = plsc.VectorSubcoreMesh(core_axis_name="c", subcore_axis_name="s")

    @functools.partial(
        pl.kernel,
        mesh=mesh,
        out_type=[
            jax.ShapeDtypeStruct((2 * E_PAD, D), jnp.float32),
            jax.ShapeDtypeStruct((2 * E_PAD, 2 * D), jnp.float32),
        ],
        scratch_types=[
            pltpu.VMEM((PER_W,), jnp.int32),
            pltpu.VMEM((PER_W,), jnp.int32),
            pltpu.VMEM((CHUNK, D), jnp.float32),
            pltpu.VMEM((CHUNK, 2 * D), jnp.float32),
            pltpu.SemaphoreType.DMA,
            pltpu.SemaphoreType.DMA,
        ],
    )
    def g(q_hbm, kv_hbm, dstg_hbm, srcg_hbm, qd_hbm, kvs_hbm,
          idxd_v, idxs_v, q_v, kv_v, semq, semkv):
        wid = lax.axis_index("s") * 2 + lax.axis_index("c")
        tbase = wid * PER_W
        pltpu.sync_copy(dstg_hbm.at[pl.ds(tbase, PER_W)], idxd_v)
        pltpu.sync_copy(srcg_hbm.at[pl.ds(tbase, PER_W)], idxs_v)

        def body(ci, carry):
            b = ci * CHUNK
            cpq = pltpu.async_copy(q_hbm.at[idxd_v.at[pl.ds(b, CHUNK)]], q_v, semq)
            cpkv = pltpu.async_copy(kv_hbm.at[idxs_v.at[pl.ds(b, CHUNK)]], kv_v, semkv)
            cpq.wait()
            cpkv.wait()
            pltpu.sync_copy(q_v, qd_hbm.at[pl.ds(tbase + b, CHUNK)])
            pltpu.sync_copy(kv_v, kvs_hbm.at[pl.ds(tbase + b, CHUNK)])
            return carry

        lax.fori_loop(0, PER_W // CHUNK, body, 0)

    return g(q_tab, kv_tab, dst_g, src_g)


def _p2_body(qd_ref, kvs_ref, msg_ref, e_ref):
    blk = qd_ref.shape[0]
    qd = qd_ref[...]
    ks = kvs_ref[:, :D]
    vs = kvs_ref[:, D:]
    ri = lax.broadcasted_iota(jnp.int32, (D, H), 0) // DK
    ci = lax.broadcasted_iota(jnp.int32, (D, H), 1)
    sel = (ri == ci).astype(jnp.float32)
    t = jnp.dot(qd * ks, sel, preferred_element_type=jnp.float32)
    rows = pl.program_id(0) * blk + lax.broadcasted_iota(jnp.int32, (blk, 1), 0)
    r2 = jnp.where(rows >= E_PAD, rows - E_PAD, rows)
    e = jnp.exp(t) * (r2 < E).astype(jnp.float32)
    e_rep = jnp.dot(e, sel.T, preferred_element_type=jnp.float32)
    msg_ref[...] = vs * e_rep
    e_ref[...] = e_rep


def _p2(qd, kvs, interpret=False):
    blk = 1280
    grid = ((2 * E_PAD) // blk,)
    return pl.pallas_call(
        _p2_body,
        grid=grid,
        in_specs=[
            pl.BlockSpec((blk, D), lambda i: (i, 0)),
            pl.BlockSpec((blk, 2 * D), lambda i: (i, 0)),
        ],
        out_specs=[
            pl.BlockSpec((blk, D), lambda i: (i, 0)),
            pl.BlockSpec((blk, D), lambda i: (i, 0)),
        ],
        out_shape=[
            jax.ShapeDtypeStruct((2 * E_PAD, D), jnp.float32),
            jax.ShapeDtypeStruct((2 * E_PAD, D), jnp.float32),
        ],
        interpret=interpret,
    )(qd, kvs)


def _sc_scatter_one(vals, dst_l, zeros, width):
    mesh = plsc.VectorSubcoreMesh(core_axis_name="c", subcore_axis_name="s")

    @functools.partial(
        pl.kernel,
        mesh=mesh,
        out_type=jax.ShapeDtypeStruct((N_ALL, width), jnp.float32),
        scratch_types=[
            pltpu.VMEM((CHUNK,), jnp.int32),
            pltpu.VMEM((CHUNK, width), jnp.float32),
            pltpu.VMEM_SHARED((N_USER, width), jnp.float32),
        ],
    )
    def s(vals_hbm, dstl_hbm, z_hbm, agg_hbm, idx_v, v_v, acc_sh):
        cid = lax.axis_index("c")
        sid = lax.axis_index("s")
        rb = sid * ROWS_A

        @pl.when(sid < N_SUB - 1)
        def _init_a():
            pltpu.sync_copy(z_hbm.at[pl.ds(rb, ROWS_A)], acc_sh.at[pl.ds(rb, ROWS_A)])

        @pl.when(sid == N_SUB - 1)
        def _init_b():
            pltpu.sync_copy(z_hbm.at[pl.ds(RB_LAST, ROWS_B)],
                            acc_sh.at[pl.ds(RB_LAST, ROWS_B)])

        plsc.subcore_barrier()
        ebase = cid * E_PAD + sid * PER_T

        def body(ci, carry):
            b = ci * CHUNK
            pltpu.sync_copy(dstl_hbm.at[pl.ds(ebase + b, CHUNK)], idx_v)
            pltpu.sync_copy(vals_hbm.at[pl.ds(ebase + b, CHUNK)], v_v)
            pltpu.sync_copy(v_v, acc_sh.at[idx_v], add=True)
            return carry

        lax.fori_loop(0, PER_T // CHUNK, body, 0)
        plsc.subcore_barrier()
        no = (1 - cid) * N_USER

        @pl.when(sid < N_SUB - 1)
        def _dump_a():
            pltpu.sync_copy(acc_sh.at[pl.ds(rb, ROWS_A)],
                            agg_hbm.at[pl.ds(no + rb, ROWS_A)])

        @pl.when(sid == N_SUB - 1)
        def _dump_b():
            pltpu.sync_copy(acc_sh.at[pl.ds(RB_LAST, ROWS_B)],
                            agg_hbm.at[pl.ds(no + RB_LAST, ROWS_B)])

    return s(vals, dst_l, zeros)


def _sc_scatter(msg, e_rep, dst_l, zmsg):
    aggm = _sc_scatter_one(msg, dst_l, zmsg, D)
    aggs = _sc_scatter_one(e_rep, dst_l, zmsg, D)
    return aggm, aggs


def _p4_body(aggm_ref, aggs_ref, f_ref, aw_ref, ab_ref, al_ref, lg_ref, lb_ref, o_ref):
    s_rep = aggs_ref[...]
    agg = jnp.where(s_rep > 0.0,
                    aggm_ref[...] / jnp.where(s_rep > 0.0, s_rep, 1.0), 0.0)
    trans = jnp.dot(agg, aw_ref[0], preferred_element_type=jnp.float32) + ab_ref[0]
    alpha = al_ref[0]
    o = alpha * trans + (1.0 - alpha) * f_ref[...]
    mu = jnp.mean(o, axis=-1, keepdims=True)
    var = jnp.mean((o - mu) ** 2, axis=-1, keepdims=True)
    o_ref[...] = (o - mu) * lax.rsqrt(var + 1e-5) * lg_ref[0] + lb_ref[0]


def _p4(aggm, aggs, feats, aw, ab, alpha, lng, lnb, interpret=False):
    blk = 1000
    grid = (N_ALL // blk,)
    nt = lambda i: i // (N_USER // blk)
    return pl.pallas_call(
        _p4_body,
        grid=grid,
        in_specs=[
            pl.BlockSpec((blk, D), lambda i: (i, 0)),
            pl.BlockSpec((blk, D), lambda i: (i, 0)),
            pl.BlockSpec((blk, D), lambda i: (i, 0)),
            pl.BlockSpec((1, D, D), lambda i: (nt(i), 0, 0)),
            pl.BlockSpec((1, 1, D), lambda i: (nt(i), 0, 0)),
            pl.BlockSpec((1, 1, 1), lambda i: (nt(i), 0, 0)),
            pl.BlockSpec((1, 1, D), lambda i: (nt(i), 0, 0)),
            pl.BlockSpec((1, 1, D), lambda i: (nt(i), 0, 0)),
        ],
        out_specs=pl.BlockSpec((blk, D), lambda i: (i, 0)),
        out_shape=jax.ShapeDtypeStruct((N_ALL, D), jnp.float32),
        interpret=interpret,
    )(aggm, aggs, feats, aw, ab.reshape(2, 1, D), alpha.reshape(2, 1, 1),
      lng.reshape(2, 1, D), lnb.reshape(2, 1, D))


def kernel(feat_user, feat_item, edge_index_clicks, edge_index_clicked_by, params):
    p = params
    inv = 1.0 / math.sqrt(DK)
    wk_u, bk_u = _fold(p["k_w"]["user"], p["k_b"]["user"], p["w_att"]["clicks"],
                       p["mu"]["clicks"] * inv)
    wv_u, bv_u = _fold(p["v_w"]["user"], p["v_b"]["user"], p["w_msg"]["clicks"], None)
    wk_i, bk_i = _fold(p["k_w"]["item"], p["k_b"]["item"], p["w_att"]["clicked_by"],
                       p["mu"]["clicked_by"] * inv)
    wv_i, bv_i = _fold(p["v_w"]["item"], p["v_b"]["item"], p["w_msg"]["clicked_by"], None)
    wq = jnp.stack([p["q_w"]["user"], p["q_w"]["item"]])
    bq = jnp.stack([p["q_b"]["user"], p["q_b"]["item"]])
    wkv = jnp.stack([jnp.concatenate([wk_u, wv_u], axis=1),
                     jnp.concatenate([wk_i, wv_i], axis=1)])
    bkv = jnp.stack([jnp.concatenate([bk_u, bv_u]), jnp.concatenate([bk_i, bv_i])])
    feats = jnp.concatenate([feat_user, feat_item], axis=0)

    pad = E_PAD - E
    src_c = jnp.pad(edge_index_clicks[0], (0, pad))
    dst_c = jnp.pad(edge_index_clicks[1], (0, pad))
    src_b = jnp.pad(edge_index_clicked_by[0], (0, pad))
    dst_b = jnp.pad(edge_index_clicked_by[1], (0, pad))
    src_g = jnp.concatenate([src_c, src_b + N_USER])
    dst_g = jnp.concatenate([dst_c + N_USER, dst_b])
    dst_l = jnp.concatenate([dst_c, dst_b])

    q_tab, kv_tab = _p1(feats, wq, wkv, bq, bkv)
    qd, kvs = _sc_gather(q_tab, kv_tab, dst_g, src_g)
    msg, e_rep = _p2(qd, kvs)
    zmsg = jnp.zeros((N_USER, D), jnp.float32)
    aggm, aggs = _sc_scatter(msg, e_rep, dst_l, zmsg)

    aw = jnp.stack([p["a_w"]["user"], p["a_w"]["item"]])
    ab = jnp.stack([p["a_b"]["user"], p["a_b"]["item"]])
    alpha = jax.nn.sigmoid(jnp.stack([p["skip"]["user"], p["skip"]["item"]]))
    lng = jnp.stack([p["ln_g"]["user"], p["ln_g"]["item"]])
    lnb = jnp.stack([p["ln_b"]["user"], p["ln_b"]["item"]])
    return _p4(aggm, aggs, feats, aw, ab, alpha, lng, lnb)


# speedup vs baseline: 29.0360x; 1.0421x over previous
import functools
import math

import jax
import jax.numpy as jnp
from jax import lax
from jax.experimental import pallas as pl
from jax.experimental.pallas import tpu as pltpu
from jax.experimental.pallas import tpu_sc as plsc

N_USER = 10000
N_ITEM = 10000
N_ALL = N_USER + N_ITEM
E = 160000
D = 128
H = 8
DK = 16
E_PAD = 163840
CHUNK = 128
PER_W = (2 * E_PAD) // 32
PER_T = E_PAD // 16
N_SUB = 16
ROWS_A = 624
ROWS_B = 640
RB_LAST = ROWS_A * (N_SUB - 1)


def _fold(w, b, wh, scale):
    wf = jnp.einsum("nhi,hij->nhj", w.reshape(D, H, DK), wh)
    bf = jnp.einsum("hi,hij->hj", b.reshape(H, DK), wh)
    if scale is not None:
        wf = wf * scale[None, :, None]
        bf = bf * scale[:, None]
    return wf.reshape(D, D), bf.reshape(D)


def _p1_body(f_ref, wq_ref, wkv_ref, bq_ref, bkv_ref, q_ref, kv_ref):
    f = f_ref[...]
    q_ref[...] = jnp.dot(f, wq_ref[0], preferred_element_type=jnp.float32) + bq_ref[0]
    kv_ref[...] = jnp.dot(f, wkv_ref[0], preferred_element_type=jnp.float32) + bkv_ref[0]


def _p1(feats, wq, wkv, bq, bkv, interpret=False):
    blk = 1000
    grid = (N_ALL // blk,)
    nt = lambda i: i // (N_USER // blk)
    return pl.pallas_call(
        _p1_body,
        grid=grid,
        in_specs=[
            pl.BlockSpec((blk, D), lambda i: (i, 0)),
            pl.BlockSpec((1, D, D), lambda i: (nt(i), 0, 0)),
            pl.BlockSpec((1, D, 2 * D), lambda i: (nt(i), 0, 0)),
            pl.BlockSpec((1, 1, D), lambda i: (nt(i), 0, 0)),
            pl.BlockSpec((1, 1, 2 * D), lambda i: (nt(i), 0, 0)),
        ],
        out_specs=[
            pl.BlockSpec((blk, D), lambda i: (i, 0)),
            pl.BlockSpec((blk, 2 * D), lambda i: (i, 0)),
        ],
        out_shape=[
            jax.ShapeDtypeStruct((N_ALL, D), jnp.float32),
            jax.ShapeDtypeStruct((N_ALL, 2 * D), jnp.float32),
        ],
        interpret=interpret,
    )(feats, wq, wkv, bq.reshape(2, 1, D), bkv.reshape(2, 1, 2 * D))


def _sc_gather(q_tab, kv_tab, dst_g, src_g):
    mesh = plsc.VectorSubcoreMesh(core_axis_name="c", subcore_axis_name="s")

    @functools.partial(
        pl.kernel,
        mesh=mesh,
        out_type=[
            jax.ShapeDtypeStruct((2 * E_PAD, D), jnp.float32),
            jax.ShapeDtypeStruct((2 * E_PAD, 2 * D), jnp.float32),
        ],
        scratch_types=[
            pltpu.VMEM((PER_W,), jnp.int32),
            pltpu.VMEM((PER_W,), jnp.int32),
            pltpu.VMEM((CHUNK, D), jnp.float32),
            pltpu.VMEM((CHUNK, D), jnp.float32),
            pltpu.VMEM((CHUNK, 2 * D), jnp.float32),
            pltpu.VMEM((CHUNK, 2 * D), jnp.float32),
            pltpu.SemaphoreType.DMA,
            pltpu.SemaphoreType.DMA,
        ],
    )
    def g(q_hbm, kv_hbm, dstg_hbm, srcg_hbm, qd_hbm, kvs_hbm,
          idxd_v, idxs_v, q0, q1, kv0, kv1, sem0, sem1):
        wid = lax.axis_index("s") * 2 + lax.axis_index("c")
        tbase = wid * PER_W
        pltpu.sync_copy(dstg_hbm.at[pl.ds(tbase, PER_W)], idxd_v)
        pltpu.sync_copy(srcg_hbm.at[pl.ds(tbase, PER_W)], idxs_v)
        nc = PER_W // CHUNK

        def fire(c, bq, bkv, sem):
            b = c * CHUNK
            pltpu.async_copy(q_hbm.at[idxd_v.at[pl.ds(b, CHUNK)]], bq, sem)
            pltpu.async_copy(kv_hbm.at[idxs_v.at[pl.ds(b, CHUNK)]], bkv, sem)

        def drain(bq, bkv, sem):
            pltpu.make_async_copy(qd_hbm.at[pl.ds(0, CHUNK)], bq, sem).wait()
            pltpu.make_async_copy(kvs_hbm.at[pl.ds(0, CHUNK)], bkv, sem).wait()

        def write(c, bq, bkv):
            b = tbase + c * CHUNK
            pltpu.sync_copy(bq, qd_hbm.at[pl.ds(b, CHUNK)])
            pltpu.sync_copy(bkv, kvs_hbm.at[pl.ds(b, CHUNK)])

        fire(0, q0, kv0, sem0)

        def body(p, carry):
            c0 = 2 * p
            fire(c0 + 1, q1, kv1, sem1)
            drain(q0, kv0, sem0)
            write(c0, q0, kv0)
            fire(c0 + 2, q0, kv0, sem0)
            drain(q1, kv1, sem1)
            write(c0 + 1, q1, kv1)
            return carry

        lax.fori_loop(0, nc // 2 - 1, body, 0)
        fire(nc - 1, q1, kv1, sem1)
        drain(q0, kv0, sem0)
        write(nc - 2, q0, kv0)
        drain(q1, kv1, sem1)
        write(nc - 1, q1, kv1)

    return g(q_tab, kv_tab, dst_g, src_g)


def _p2_body(qd_ref, kvs_ref, msg_ref, e_ref):
    blk = qd_ref.shape[0]
    qd = qd_ref[...]
    ks = kvs_ref[:, :D]
    vs = kvs_ref[:, D:]
    ri = lax.broadcasted_iota(jnp.int32, (D, H), 0) // DK
    ci = lax.broadcasted_iota(jnp.int32, (D, H), 1)
    sel = (ri == ci).astype(jnp.float32)
    t = jnp.dot(qd * ks, sel, preferred_element_type=jnp.float32)
    rows = pl.program_id(0) * blk + lax.broadcasted_iota(jnp.int32, (blk, 1), 0)
    r2 = jnp.where(rows >= E_PAD, rows - E_PAD, rows)
    e = jnp.exp(t) * (r2 < E).astype(jnp.float32)
    e_rep = jnp.dot(e, sel.T, preferred_element_type=jnp.float32)
    msg_ref[...] = vs * e_rep
    e_ref[...] = e_rep


def _p2(qd, kvs, interpret=False):
    blk = 1280
    grid = ((2 * E_PAD) // blk,)
    return pl.pallas_call(
        _p2_body,
        grid=grid,
        in_specs=[
            pl.BlockSpec((blk, D), lambda i: (i, 0)),
            pl.BlockSpec((blk, 2 * D), lambda i: (i, 0)),
        ],
        out_specs=[
            pl.BlockSpec((blk, D), lambda i: (i, 0)),
            pl.BlockSpec((blk, D), lambda i: (i, 0)),
        ],
        out_shape=[
            jax.ShapeDtypeStruct((2 * E_PAD, D), jnp.float32),
            jax.ShapeDtypeStruct((2 * E_PAD, D), jnp.float32),
        ],
        interpret=interpret,
    )(qd, kvs)


def _sc_scatter_one(vals, dst_l, zeros, width):
    mesh = plsc.VectorSubcoreMesh(core_axis_name="c", subcore_axis_name="s")

    @functools.partial(
        pl.kernel,
        mesh=mesh,
        out_type=jax.ShapeDtypeStruct((N_ALL, width), jnp.float32),
        scratch_types=[
            pltpu.VMEM((CHUNK,), jnp.int32),
            pltpu.VMEM((CHUNK, width), jnp.float32),
            pltpu.VMEM_SHARED((N_USER, width), jnp.float32),
        ],
    )
    def s(vals_hbm, dstl_hbm, z_hbm, agg_hbm, idx_v, v_v, acc_sh):
        cid = lax.axis_index("c")
        sid = lax.axis_index("s")
        rb = sid * ROWS_A

        @pl.when(sid < N_SUB - 1)
        def _init_a():
            pltpu.sync_copy(z_hbm.at[pl.ds(rb, ROWS_A)], acc_sh.at[pl.ds(rb, ROWS_A)])

        @pl.when(sid == N_SUB - 1)
        def _init_b():
            pltpu.sync_copy(z_hbm.at[pl.ds(RB_LAST, ROWS_B)],
                            acc_sh.at[pl.ds(RB_LAST, ROWS_B)])

        plsc.subcore_barrier()
        ebase = cid * E_PAD + sid * PER_T

        def body(ci, carry):
            b = ci * CHUNK
            pltpu.sync_copy(dstl_hbm.at[pl.ds(ebase + b, CHUNK)], idx_v)
            pltpu.sync_copy(vals_hbm.at[pl.ds(ebase + b, CHUNK)], v_v)
            pltpu.sync_copy(v_v, acc_sh.at[idx_v], add=True)
            return carry

        lax.fori_loop(0, PER_T // CHUNK, body, 0)
        plsc.subcore_barrier()
        no = (1 - cid) * N_USER

        @pl.when(sid < N_SUB - 1)
        def _dump_a():
            pltpu.sync_copy(acc_sh.at[pl.ds(rb, ROWS_A)],
                            agg_hbm.at[pl.ds(no + rb, ROWS_A)])

        @pl.when(sid == N_SUB - 1)
        def _dump_b():
            pltpu.sync_copy(acc_sh.at[pl.ds(RB_LAST, ROWS_B)],
                            agg_hbm.at[pl.ds(no + RB_LAST, ROWS_B)])

    return s(vals, dst_l, zeros)


def _sc_scatter(msg, e_rep, dst_l, zmsg):
    aggm = _sc_scatter_one(msg, dst_l, zmsg, D)
    aggs = _sc_scatter_one(e_rep, dst_l, zmsg, D)
    return aggm, aggs


def _p4_body(aggm_ref, aggs_ref, f_ref, aw_ref, ab_ref, al_ref, lg_ref, lb_ref, o_ref):
    s_rep = aggs_ref[...]
    agg = jnp.where(s_rep > 0.0,
                    aggm_ref[...] / jnp.where(s_rep > 0.0, s_rep, 1.0), 0.0)
    trans = jnp.dot(agg, aw_ref[0], preferred_element_type=jnp.float32) + ab_ref[0]
    alpha = al_ref[0]
    o = alpha * trans + (1.0 - alpha) * f_ref[...]
    mu = jnp.mean(o, axis=-1, keepdims=True)
    var = jnp.mean((o - mu) ** 2, axis=-1, keepdims=True)
    o_ref[...] = (o - mu) * lax.rsqrt(var + 1e-5) * lg_ref[0] + lb_ref[0]


def _p4(aggm, aggs, feats, aw, ab, alpha, lng, lnb, interpret=False):
    blk = 1000
    grid = (N_ALL // blk,)
    nt = lambda i: i // (N_USER // blk)
    return pl.pallas_call(
        _p4_body,
        grid=grid,
        in_specs=[
            pl.BlockSpec((blk, D), lambda i: (i, 0)),
            pl.BlockSpec((blk, D), lambda i: (i, 0)),
            pl.BlockSpec((blk, D), lambda i: (i, 0)),
            pl.BlockSpec((1, D, D), lambda i: (nt(i), 0, 0)),
            pl.BlockSpec((1, 1, D), lambda i: (nt(i), 0, 0)),
            pl.BlockSpec((1, 1, 1), lambda i: (nt(i), 0, 0)),
            pl.BlockSpec((1, 1, D), lambda i: (nt(i), 0, 0)),
            pl.BlockSpec((1, 1, D), lambda i: (nt(i), 0, 0)),
        ],
        out_specs=pl.BlockSpec((blk, D), lambda i: (i, 0)),
        out_shape=jax.ShapeDtypeStruct((N_ALL, D), jnp.float32),
        interpret=interpret,
    )(aggm, aggs, feats, aw, ab.reshape(2, 1, D), alpha.reshape(2, 1, 1),
      lng.reshape(2, 1, D), lnb.reshape(2, 1, D))


def kernel(feat_user, feat_item, edge_index_clicks, edge_index_clicked_by, params):
    p = params
    inv = 1.0 / math.sqrt(DK)
    wk_u, bk_u = _fold(p["k_w"]["user"], p["k_b"]["user"], p["w_att"]["clicks"],
                       p["mu"]["clicks"] * inv)
    wv_u, bv_u = _fold(p["v_w"]["user"], p["v_b"]["user"], p["w_msg"]["clicks"], None)
    wk_i, bk_i = _fold(p["k_w"]["item"], p["k_b"]["item"], p["w_att"]["clicked_by"],
                       p["mu"]["clicked_by"] * inv)
    wv_i, bv_i = _fold(p["v_w"]["item"], p["v_b"]["item"], p["w_msg"]["clicked_by"], None)
    wq = jnp.stack([p["q_w"]["user"], p["q_w"]["item"]])
    bq = jnp.stack([p["q_b"]["user"], p["q_b"]["item"]])
    wkv = jnp.stack([jnp.concatenate([wk_u, wv_u], axis=1),
                     jnp.concatenate([wk_i, wv_i], axis=1)])
    bkv = jnp.stack([jnp.concatenate([bk_u, bv_u]), jnp.concatenate([bk_i, bv_i])])
    feats = jnp.concatenate([feat_user, feat_item], axis=0)

    pad = E_PAD - E
    src_c = jnp.pad(edge_index_clicks[0], (0, pad))
    dst_c = jnp.pad(edge_index_clicks[1], (0, pad))
    src_b = jnp.pad(edge_index_clicked_by[0], (0, pad))
    dst_b = jnp.pad(edge_index_clicked_by[1], (0, pad))
    src_g = jnp.concatenate([src_c, src_b + N_USER])
    dst_g = jnp.concatenate([dst_c + N_USER, dst_b])
    dst_l = jnp.concatenate([dst_c, dst_b])

    q_tab, kv_tab = _p1(feats, wq, wkv, bq, bkv)
    qd, kvs = _sc_gather(q_tab, kv_tab, dst_g, src_g)
    msg, e_rep = _p2(qd, kvs)
    zmsg = jnp.zeros((N_USER, D), jnp.float32)
    aggm, aggs = _sc_scatter(msg, e_rep, dst_l, zmsg)

    aw = jnp.stack([p["a_w"]["user"], p["a_w"]["item"]])
    ab = jnp.stack([p["a_b"]["user"], p["a_b"]["item"]])
    alpha = jax.nn.sigmoid(jnp.stack([p["skip"]["user"], p["skip"]["item"]]))
    lng = jnp.stack([p["ln_g"]["user"], p["ln_g"]["item"]])
    lnb = jnp.stack([p["ln_b"]["user"], p["ln_b"]["item"]])
    return _p4(aggm, aggs, feats, aw, ab, alpha, lng, lnb)


# speedup vs baseline: 33.9192x; 1.1682x over previous
import functools
import math

import jax
import jax.numpy as jnp
from jax import lax
from jax.experimental import pallas as pl
from jax.experimental.pallas import tpu as pltpu
from jax.experimental.pallas import tpu_sc as plsc

N_USER = 10000
N_ITEM = 10000
N_ALL = N_USER + N_ITEM
E = 160000
D = 128
H = 8
DK = 16
E_PAD = 163840
CHUNK = 128
PER_W = (2 * E_PAD) // 32
PER_T = E_PAD // 16
N_SUB = 16
ROWS_A = 624
ROWS_B = 640
RB_LAST = ROWS_A * (N_SUB - 1)


def _fold(w, b, wh, scale):
    wf = jnp.einsum("nhi,hij->nhj", w.reshape(D, H, DK), wh)
    bf = jnp.einsum("hi,hij->hj", b.reshape(H, DK), wh)
    if scale is not None:
        wf = wf * scale[None, :, None]
        bf = bf * scale[:, None]
    return wf.reshape(D, D), bf.reshape(D)


def _p1_body(f_ref, wq_ref, wkv_ref, bq_ref, bkv_ref, q_ref, kv_ref):
    f = f_ref[...]
    q_ref[...] = jnp.dot(f, wq_ref[0], preferred_element_type=jnp.float32) + bq_ref[0]
    kv_ref[...] = jnp.dot(f, wkv_ref[0], preferred_element_type=jnp.float32) + bkv_ref[0]


def _p1(feats, wq, wkv, bq, bkv, interpret=False):
    blk = 1000
    grid = (N_ALL // blk,)
    nt = lambda i: i // (N_USER // blk)
    return pl.pallas_call(
        _p1_body,
        grid=grid,
        in_specs=[
            pl.BlockSpec((blk, D), lambda i: (i, 0)),
            pl.BlockSpec((1, D, D), lambda i: (nt(i), 0, 0)),
            pl.BlockSpec((1, D, 2 * D), lambda i: (nt(i), 0, 0)),
            pl.BlockSpec((1, 1, D), lambda i: (nt(i), 0, 0)),
            pl.BlockSpec((1, 1, 2 * D), lambda i: (nt(i), 0, 0)),
        ],
        out_specs=[
            pl.BlockSpec((blk, D), lambda i: (i, 0)),
            pl.BlockSpec((blk, 2 * D), lambda i: (i, 0)),
        ],
        out_shape=[
            jax.ShapeDtypeStruct((N_ALL, D), jnp.float32),
            jax.ShapeDtypeStruct((N_ALL, 2 * D), jnp.float32),
        ],
        interpret=interpret,
    )(feats, wq, wkv, bq.reshape(2, 1, D), bkv.reshape(2, 1, 2 * D))


def _sc_gather(q_tab, kv_tab, dst_g, src_g):
    mesh = plsc.VectorSubcoreMesh(core_axis_name="c", subcore_axis_name="s")

    @functools.partial(
        pl.kernel,
        mesh=mesh,
        out_type=[
            jax.ShapeDtypeStruct((2 * E_PAD, D), jnp.float32),
            jax.ShapeDtypeStruct((2 * E_PAD, 2 * D), jnp.float32),
        ],
        scratch_types=[
            pltpu.VMEM((PER_W,), jnp.int32),
            pltpu.VMEM((PER_W,), jnp.int32),
            pltpu.VMEM((CHUNK, D), jnp.float32),
            pltpu.VMEM((CHUNK, D), jnp.float32),
            pltpu.VMEM((CHUNK, 2 * D), jnp.float32),
            pltpu.VMEM((CHUNK, 2 * D), jnp.float32),
            pltpu.SemaphoreType.DMA,
            pltpu.SemaphoreType.DMA,
        ],
    )
    def g(q_hbm, kv_hbm, dstg_hbm, srcg_hbm, qd_hbm, kvs_hbm,
          idxd_v, idxs_v, q0, q1, kv0, kv1, sem0, sem1):
        wid = lax.axis_index("s") * 2 + lax.axis_index("c")
        tbase = wid * PER_W
        pltpu.sync_copy(dstg_hbm.at[pl.ds(tbase, PER_W)], idxd_v)
        pltpu.sync_copy(srcg_hbm.at[pl.ds(tbase, PER_W)], idxs_v)
        nc = PER_W // CHUNK

        def fire(c, bq, bkv, sem):
            b = c * CHUNK
            pltpu.async_copy(q_hbm.at[idxd_v.at[pl.ds(b, CHUNK)]], bq, sem)
            pltpu.async_copy(kv_hbm.at[idxs_v.at[pl.ds(b, CHUNK)]], bkv, sem)

        def drain(bq, bkv, sem):
            pltpu.make_async_copy(qd_hbm.at[pl.ds(0, CHUNK)], bq, sem).wait()
            pltpu.make_async_copy(kvs_hbm.at[pl.ds(0, CHUNK)], bkv, sem).wait()

        def write(c, bq, bkv):
            b = tbase + c * CHUNK
            pltpu.sync_copy(bq, qd_hbm.at[pl.ds(b, CHUNK)])
            pltpu.sync_copy(bkv, kvs_hbm.at[pl.ds(b, CHUNK)])

        fire(0, q0, kv0, sem0)

        def body(p, carry):
            c0 = 2 * p
            fire(c0 + 1, q1, kv1, sem1)
            drain(q0, kv0, sem0)
            write(c0, q0, kv0)
            fire(c0 + 2, q0, kv0, sem0)
            drain(q1, kv1, sem1)
            write(c0 + 1, q1, kv1)
            return carry

        lax.fori_loop(0, nc // 2 - 1, body, 0)
        fire(nc - 1, q1, kv1, sem1)
        drain(q0, kv0, sem0)
        write(nc - 2, q0, kv0)
        drain(q1, kv1, sem1)
        write(nc - 1, q1, kv1)

    return g(q_tab, kv_tab, dst_g, src_g)


def _p2_body(qd_ref, kvs_ref, msg_ref, e_ref):
    blk = qd_ref.shape[0]
    qd = qd_ref[...]
    ks = kvs_ref[:, :D]
    vs = kvs_ref[:, D:]
    ri = lax.broadcasted_iota(jnp.int32, (D, H), 0) // DK
    ci = lax.broadcasted_iota(jnp.int32, (D, H), 1)
    sel = (ri == ci).astype(jnp.float32)
    t = jnp.dot(qd * ks, sel, preferred_element_type=jnp.float32)
    rows = pl.program_id(0) * blk + lax.broadcasted_iota(jnp.int32, (blk, 1), 0)
    r2 = jnp.where(rows >= E_PAD, rows - E_PAD, rows)
    e = jnp.exp(t) * (r2 < E).astype(jnp.float32)
    e_rep = jnp.dot(e, sel.T, preferred_element_type=jnp.float32)
    msg_ref[...] = vs * e_rep
    e_ref[...] = e_rep


def _p2(qd, kvs, interpret=False):
    blk = 1280
    grid = ((2 * E_PAD) // blk,)
    return pl.pallas_call(
        _p2_body,
        grid=grid,
        in_specs=[
            pl.BlockSpec((blk, D), lambda i: (i, 0)),
            pl.BlockSpec((blk, 2 * D), lambda i: (i, 0)),
        ],
        out_specs=[
            pl.BlockSpec((blk, D), lambda i: (i, 0)),
            pl.BlockSpec((blk, D), lambda i: (i, 0)),
        ],
        out_shape=[
            jax.ShapeDtypeStruct((2 * E_PAD, D), jnp.float32),
            jax.ShapeDtypeStruct((2 * E_PAD, D), jnp.float32),
        ],
        interpret=interpret,
    )(qd, kvs)


def _sc_scatter_one(vals, dst_l, zeros, width):
    mesh = plsc.VectorSubcoreMesh(core_axis_name="c", subcore_axis_name="s")

    @functools.partial(
        pl.kernel,
        mesh=mesh,
        out_type=jax.ShapeDtypeStruct((N_ALL, width), jnp.float32),
        scratch_types=[
            pltpu.VMEM((CHUNK,), jnp.int32),
            pltpu.VMEM((CHUNK,), jnp.int32),
            pltpu.VMEM((CHUNK, width), jnp.float32),
            pltpu.VMEM((CHUNK, width), jnp.float32),
            pltpu.VMEM_SHARED((N_USER, width), jnp.float32),
            pltpu.SemaphoreType.DMA,
            pltpu.SemaphoreType.DMA,
        ],
    )
    def s(vals_hbm, dstl_hbm, z_hbm, agg_hbm, i0, i1, v0, v1, acc_sh, sem0, sem1):
        cid = lax.axis_index("c")
        sid = lax.axis_index("s")
        rb = sid * ROWS_A

        @pl.when(sid < N_SUB - 1)
        def _init_a():
            pltpu.sync_copy(z_hbm.at[pl.ds(rb, ROWS_A)], acc_sh.at[pl.ds(rb, ROWS_A)])

        @pl.when(sid == N_SUB - 1)
        def _init_b():
            pltpu.sync_copy(z_hbm.at[pl.ds(RB_LAST, ROWS_B)],
                            acc_sh.at[pl.ds(RB_LAST, ROWS_B)])

        plsc.subcore_barrier()
        ebase = cid * E_PAD + sid * PER_T
        nc = PER_T // CHUNK

        def fire(c, bi, bv, sem):
            b = ebase + c * CHUNK
            pltpu.async_copy(dstl_hbm.at[pl.ds(b, CHUNK)], bi, sem)
            pltpu.async_copy(vals_hbm.at[pl.ds(b, CHUNK)], bv, sem)

        def drain(bi, bv, sem):
            pltpu.make_async_copy(dstl_hbm.at[pl.ds(0, CHUNK)], bi, sem).wait()
            pltpu.make_async_copy(vals_hbm.at[pl.ds(0, CHUNK)], bv, sem).wait()

        def scat(bi, bv):
            pltpu.sync_copy(bv, acc_sh.at[bi], add=True)

        fire(0, i0, v0, sem0)

        def body(p, carry):
            c0 = 2 * p
            fire(c0 + 1, i1, v1, sem1)
            drain(i0, v0, sem0)
            scat(i0, v0)
            fire(c0 + 2, i0, v0, sem0)
            drain(i1, v1, sem1)
            scat(i1, v1)
            return carry

        lax.fori_loop(0, nc // 2 - 1, body, 0)
        fire(nc - 1, i1, v1, sem1)
        drain(i0, v0, sem0)
        scat(i0, v0)
        drain(i1, v1, sem1)
        scat(i1, v1)
        plsc.subcore_barrier()
        no = (1 - cid) * N_USER

        @pl.when(sid < N_SUB - 1)
        def _dump_a():
            pltpu.sync_copy(acc_sh.at[pl.ds(rb, ROWS_A)],
                            agg_hbm.at[pl.ds(no + rb, ROWS_A)])

        @pl.when(sid == N_SUB - 1)
        def _dump_b():
            pltpu.sync_copy(acc_sh.at[pl.ds(RB_LAST, ROWS_B)],
                            agg_hbm.at[pl.ds(no + RB_LAST, ROWS_B)])

    return s(vals, dst_l, zeros)


def _sc_scatter(msg, e_rep, dst_l, zmsg):
    aggm = _sc_scatter_one(msg, dst_l, zmsg, D)
    aggs = _sc_scatter_one(e_rep, dst_l, zmsg, D)
    return aggm, aggs


def _p4_body(aggm_ref, aggs_ref, f_ref, aw_ref, ab_ref, al_ref, lg_ref, lb_ref, o_ref):
    s_rep = aggs_ref[...]
    agg = jnp.where(s_rep > 0.0,
                    aggm_ref[...] / jnp.where(s_rep > 0.0, s_rep, 1.0), 0.0)
    trans = jnp.dot(agg, aw_ref[0], preferred_element_type=jnp.float32) + ab_ref[0]
    alpha = al_ref[0]
    o = alpha * trans + (1.0 - alpha) * f_ref[...]
    mu = jnp.mean(o, axis=-1, keepdims=True)
    var = jnp.mean((o - mu) ** 2, axis=-1, keepdims=True)
    o_ref[...] = (o - mu) * lax.rsqrt(var + 1e-5) * lg_ref[0] + lb_ref[0]


def _p4(aggm, aggs, feats, aw, ab, alpha, lng, lnb, interpret=False):
    blk = 1000
    grid = (N_ALL // blk,)
    nt = lambda i: i // (N_USER // blk)
    return pl.pallas_call(
        _p4_body,
        grid=grid,
        in_specs=[
            pl.BlockSpec((blk, D), lambda i: (i, 0)),
            pl.BlockSpec((blk, D), lambda i: (i, 0)),
            pl.BlockSpec((blk, D), lambda i: (i, 0)),
            pl.BlockSpec((1, D, D), lambda i: (nt(i), 0, 0)),
            pl.BlockSpec((1, 1, D), lambda i: (nt(i), 0, 0)),
            pl.BlockSpec((1, 1, 1), lambda i: (nt(i), 0, 0)),
            pl.BlockSpec((1, 1, D), lambda i: (nt(i), 0, 0)),
            pl.BlockSpec((1, 1, D), lambda i: (nt(i), 0, 0)),
        ],
        out_specs=pl.BlockSpec((blk, D), lambda i: (i, 0)),
        out_shape=jax.ShapeDtypeStruct((N_ALL, D), jnp.float32),
        interpret=interpret,
    )(aggm, aggs, feats, aw, ab.reshape(2, 1, D), alpha.reshape(2, 1, 1),
      lng.reshape(2, 1, D), lnb.reshape(2, 1, D))


def kernel(feat_user, feat_item, edge_index_clicks, edge_index_clicked_by, params):
    p = params
    inv = 1.0 / math.sqrt(DK)
    wk_u, bk_u = _fold(p["k_w"]["user"], p["k_b"]["user"], p["w_att"]["clicks"],
                       p["mu"]["clicks"] * inv)
    wv_u, bv_u = _fold(p["v_w"]["user"], p["v_b"]["user"], p["w_msg"]["clicks"], None)
    wk_i, bk_i = _fold(p["k_w"]["item"], p["k_b"]["item"], p["w_att"]["clicked_by"],
                       p["mu"]["clicked_by"] * inv)
    wv_i, bv_i = _fold(p["v_w"]["item"], p["v_b"]["item"], p["w_msg"]["clicked_by"], None)
    wq = jnp.stack([p["q_w"]["user"], p["q_w"]["item"]])
    bq = jnp.stack([p["q_b"]["user"], p["q_b"]["item"]])
    wkv = jnp.stack([jnp.concatenate([wk_u, wv_u], axis=1),
                     jnp.concatenate([wk_i, wv_i], axis=1)])
    bkv = jnp.stack([jnp.concatenate([bk_u, bv_u]), jnp.concatenate([bk_i, bv_i])])
    feats = jnp.concatenate([feat_user, feat_item], axis=0)

    pad = E_PAD - E
    src_c = jnp.pad(edge_index_clicks[0], (0, pad))
    dst_c = jnp.pad(edge_index_clicks[1], (0, pad))
    src_b = jnp.pad(edge_index_clicked_by[0], (0, pad))
    dst_b = jnp.pad(edge_index_clicked_by[1], (0, pad))
    src_g = jnp.concatenate([src_c, src_b + N_USER])
    dst_g = jnp.concatenate([dst_c + N_USER, dst_b])
    dst_l = jnp.concatenate([dst_c, dst_b])

    q_tab, kv_tab = _p1(feats, wq, wkv, bq, bkv)
    qd, kvs = _sc_gather(q_tab, kv_tab, dst_g, src_g)
    msg, e_rep = _p2(qd, kvs)
    zmsg = jnp.zeros((N_USER, D), jnp.float32)
    aggm, aggs = _sc_scatter(msg, e_rep, dst_l, zmsg)

    aw = jnp.stack([p["a_w"]["user"], p["a_w"]["item"]])
    ab = jnp.stack([p["a_b"]["user"], p["a_b"]["item"]])
    alpha = jax.nn.sigmoid(jnp.stack([p["skip"]["user"], p["skip"]["item"]]))
    lng = jnp.stack([p["ln_g"]["user"], p["ln_g"]["item"]])
    lnb = jnp.stack([p["ln_b"]["user"], p["ln_b"]["item"]])
    return _p4(aggm, aggs, feats, aw, ab, alpha, lng, lnb)
